# Initial kernel scaffold; baseline (speedup 1.0000x reference)
import jax
import jax.numpy as jnp
from jax.experimental import pallas as pl


def kernel(x, length, emb, W1, b1, W2, b2, W3, b3):
    raise NotImplementedError("write your pallas kernel here")


# speedup vs baseline: 26.7348x; 26.7348x over previous
import functools

import jax
import jax.numpy as jnp
from jax import lax
from jax.experimental import pallas as pl
from jax.experimental.pallas import tpu as pltpu
from jax.experimental.pallas import tpu_sc as plsc

B = 4096
L = 200
VOCAB = 1000
VPAD = 1024
DIM = 128

NC = 2
NS = 16
LANES = 16
NW = NC * NS
ROWS_W = B // NW
GROUPS = ROWS_W // LANES


HIST_W = LANES * VPAD


def _sc_hist_kernel(xg_hbm, counts_hbm, x_v, hist_v):
    wid = lax.axis_index("s") * NC + lax.axis_index("c")
    base = wid * ROWS_W
    pltpu.sync_copy(xg_hbm.at[wid], x_v)

    lane_off = lax.iota(jnp.int32, LANES) * VPAD
    ones = jnp.full((LANES,), 1.0, dtype=jnp.float32)
    zeros = jnp.zeros((LANES,), dtype=jnp.float32)

    for g in range(GROUPS):
        def zero_body(i, _):
            hist_v[pl.ds(i * LANES, LANES)] = zeros
            return 0

        lax.fori_loop(0, HIST_W // LANES, zero_body, 0)

        def tok_body(t, _):
            toks = x_v[pl.ds(t * ROWS_W + g * LANES, LANES)]
            plsc.addupdate_scatter(hist_v, [lane_off + toks], ones)
            return 0

        lax.fori_loop(0, L, tok_body, 0)

        pltpu.sync_copy(
            hist_v, counts_hbm.at[pl.ds((base + g * LANES) * VPAD, HIST_W)])


@functools.partial(jax.jit, static_argnames=())
def _sc_hist(xg):
    mesh = plsc.VectorSubcoreMesh(core_axis_name="c", subcore_axis_name="s")
    return pl.kernel(
        _sc_hist_kernel,
        out_type=jax.ShapeDtypeStruct((B * VPAD,), jnp.float32),
        mesh=mesh,
        scratch_types=[
            pltpu.VMEM((L * ROWS_W,), jnp.int32),
            pltpu.VMEM((HIST_W,), jnp.float32),
        ],
        compiler_params=pltpu.CompilerParams(needs_layout_passes=False),
    )(xg)


def _tc_mlp_kernel(counts_ref, emb_ref, W1_ref, b1_ref, W2_ref, b2_ref,
                   W3_ref, b3_ref, out_ref):
    m = jnp.dot(counts_ref[...], emb_ref[...],
                preferred_element_type=jnp.float32) * (1.0 / L)
    h = jnp.maximum(jnp.dot(m, W1_ref[...],
                            preferred_element_type=jnp.float32) + b1_ref[...],
                    0.0)
    h2 = jnp.dot(h, W2_ref[...], preferred_element_type=jnp.float32) + b2_ref[...]
    h2 = jnp.where(h2 >= 0, h2, 0.01 * h2)
    out_ref[...] = jnp.dot(h2, W3_ref[...],
                           preferred_element_type=jnp.float32) + b3_ref[...]


def _tc_mlp(counts, emb_pad, W1, b1, W2, b2, W3, b3):
    BM = 512
    grid = (B // BM,)
    return pl.pallas_call(
        _tc_mlp_kernel,
        grid=grid,
        in_specs=[
            pl.BlockSpec((BM, VPAD), lambda i: (i, 0)),
            pl.BlockSpec((VPAD, DIM), lambda i: (0, 0)),
            pl.BlockSpec(W1.shape, lambda i: (0, 0)),
            pl.BlockSpec(b1.shape, lambda i: (0, 0)),
            pl.BlockSpec(W2.shape, lambda i: (0, 0)),
            pl.BlockSpec(b2.shape, lambda i: (0, 0)),
            pl.BlockSpec(W3.shape, lambda i: (0, 0)),
            pl.BlockSpec(b3.shape, lambda i: (0, 0)),
        ],
        out_specs=pl.BlockSpec((BM, 2), lambda i: (i, 0)),
        out_shape=jax.ShapeDtypeStruct((B, 2), jnp.float32),
    )(counts, emb_pad, W1, b1, W2, b2, W3, b3)


def kernel(x, length, emb, W1, b1, W2, b2, W3, b3):
    del length
    xg = x.astype(jnp.int32).reshape(NW, ROWS_W, L).transpose(0, 2, 1)
    xg = xg.reshape(NW, L * ROWS_W)
    counts = _sc_hist(xg).reshape(B, VPAD)
    emb_pad = jnp.pad(emb, ((0, VPAD - VOCAB), (0, 0)))
    b1r = b1.reshape(1, -1)
    b2r = b2.reshape(1, -1)
    b3r = b3.reshape(1, -1)
    return _tc_mlp(counts, emb_pad, W1, b1r, W2, b2r, W3, b3r)


# speedup vs baseline: 38.3102x; 1.4330x over previous
import functools

import jax
import jax.numpy as jnp
from jax import lax
from jax.experimental import pallas as pl
from jax.experimental.pallas import tpu as pltpu
from jax.experimental.pallas import tpu_sc as plsc

B = 4096
L = 200
VOCAB = 1000
VPAD = 1024
DIM = 128

NC = 2
NS = 16
LANES = 16
NW = NC * NS
ROWS_W = B // NW
GROUPS = ROWS_W // LANES


HIST_W = LANES * VPAD


def _sc_hist_kernel(xg_hbm, counts_hbm, x_v, hist_v, sem0, sem1):
    wid = lax.axis_index("s") * NC + lax.axis_index("c")
    base = wid * ROWS_W
    pltpu.sync_copy(xg_hbm.at[wid], x_v)

    lane_off = lax.iota(jnp.int32, LANES) * VPAD
    ones = jnp.full((LANES,), 1.0, dtype=jnp.float32)
    zeros = jnp.zeros((LANES,), dtype=jnp.float32)
    sems = [sem0, sem1]
    pending = [None, None]

    for g in range(GROUPS):
        b = g % 2
        boff = b * HIST_W
        if pending[b] is not None:
            pending[b].wait()

        def _zero(i, _, boff=boff):
            for u in range(16):
                hist_v[pl.ds(boff + (i * 16 + u) * LANES, LANES)] = zeros
            return 0

        lax.fori_loop(0, HIST_W // LANES // 16, _zero, 0)

        lane_off_g = lane_off + boff

        def _scat(t, _, g=g, lane_off_g=lane_off_g):
            for u in range(8):
                toks = x_v[pl.ds((t * 8 + u) * ROWS_W + g * LANES, LANES)]
                plsc.addupdate_scatter(hist_v, [lane_off_g + toks], ones)
            return 0

        lax.fori_loop(0, L // 8, _scat, 0)

        pending[b] = pltpu.async_copy(
            hist_v.at[pl.ds(boff, HIST_W)],
            counts_hbm.at[pl.ds((base + g * LANES) * VPAD, HIST_W)],
            sems[b])

    pending[0].wait()
    pending[1].wait()


@functools.partial(jax.jit, static_argnames=())
def _sc_hist(xg):
    mesh = plsc.VectorSubcoreMesh(core_axis_name="c", subcore_axis_name="s")
    return pl.kernel(
        _sc_hist_kernel,
        out_type=jax.ShapeDtypeStruct((B * VPAD,), jnp.float32),
        mesh=mesh,
        scratch_types=[
            pltpu.VMEM((L * ROWS_W,), jnp.int32),
            pltpu.VMEM((2 * HIST_W,), jnp.float32),
            pltpu.SemaphoreType.DMA,
            pltpu.SemaphoreType.DMA,
        ],
        compiler_params=pltpu.CompilerParams(needs_layout_passes=False),
    )(xg)


def _tc_mlp_kernel(counts_ref, emb_ref, W1_ref, b1_ref, W2_ref, b2_ref,
                   W3_ref, b3_ref, out_ref):
    m = jnp.dot(counts_ref[...], emb_ref[...],
                preferred_element_type=jnp.float32) * (1.0 / L)
    h = jnp.maximum(jnp.dot(m, W1_ref[...],
                            preferred_element_type=jnp.float32) + b1_ref[...],
                    0.0)
    h2 = jnp.dot(h, W2_ref[...], preferred_element_type=jnp.float32) + b2_ref[...]
    h2 = jnp.where(h2 >= 0, h2, 0.01 * h2)
    out_ref[...] = jnp.dot(h2, W3_ref[...],
                           preferred_element_type=jnp.float32) + b3_ref[...]


def _tc_mlp(counts, emb_pad, W1, b1, W2, b2, W3, b3):
    BM = 512
    grid = (B // BM,)
    return pl.pallas_call(
        _tc_mlp_kernel,
        grid=grid,
        in_specs=[
            pl.BlockSpec((BM, VPAD), lambda i: (i, 0)),
            pl.BlockSpec((VPAD, DIM), lambda i: (0, 0)),
            pl.BlockSpec(W1.shape, lambda i: (0, 0)),
            pl.BlockSpec(b1.shape, lambda i: (0, 0)),
            pl.BlockSpec(W2.shape, lambda i: (0, 0)),
            pl.BlockSpec(b2.shape, lambda i: (0, 0)),
            pl.BlockSpec(W3.shape, lambda i: (0, 0)),
            pl.BlockSpec(b3.shape, lambda i: (0, 0)),
        ],
        out_specs=pl.BlockSpec((BM, 2), lambda i: (i, 0)),
        out_shape=jax.ShapeDtypeStruct((B, 2), jnp.float32),
    )(counts, emb_pad, W1, b1, W2, b2, W3, b3)


def kernel(x, length, emb, W1, b1, W2, b2, W3, b3):
    del length
    xg = x.astype(jnp.int32).reshape(NW, ROWS_W, L).transpose(0, 2, 1)
    xg = xg.reshape(NW, L * ROWS_W)
    counts = _sc_hist(xg).reshape(B, VPAD)
    emb_pad = jnp.pad(emb, ((0, VPAD - VOCAB), (0, 0)))
    b1r = b1.reshape(1, -1)
    b2r = b2.reshape(1, -1)
    b3r = b3.reshape(1, -1)
    return _tc_mlp(counts, emb_pad, W1, b1r, W2, b2r, W3, b3r)
